# baseline (speedup 1.0000x reference)
import functools

import jax
import jax.numpy as jnp
from jax import lax
from jax.experimental import pallas as pl
from jax.experimental.pallas import tpu as pltpu
from jax.experimental.pallas import tpu_sc as plsc

LS = 0.1
V = 100000
B = 1024
CONF = 1.0 - LS
SMOOTH = LS / (V - 2)
GRAN = 128
SUB = 8
V_ALIGNED = (V // GRAN) * GRAN
N_TILES = V_ALIGNED // GRAN

TC_ROWS = 768
SUM_BLK_ROWS = 16
NSTEP = TC_ROWS // SUM_BLK_ROWS
DEPTH = 8

SC_BLK_COLS = 1408
SC_COL_BLKS = V_ALIGNED // SC_BLK_COLS
SC_ROW_GROUPS = (B - TC_ROWS) // SUB
_SC_WORKERS = 32
_LANES = 16


def _sc_sum(output):
    mesh = plsc.VectorSubcoreMesh(core_axis_name="c", subcore_axis_name="s")

    @functools.partial(
        pl.kernel,
        mesh=mesh,
        out_type=jax.ShapeDtypeStruct((_SC_WORKERS, 1, _LANES), jnp.float32),
        scratch_types=[
            pltpu.VMEM((1, _LANES), jnp.float32),
        ],
    )
    def k(x_hbm, p_hbm, acc_v):
        wid = lax.axis_index("s") * 2 + lax.axis_index("c")
        acc_v[...] = jnp.zeros((1, _LANES), jnp.float32)

        def body(x_vmem):
            accs = [jnp.zeros((1, _LANES), jnp.float32) for _ in range(4)]
            n = 0
            for r in range(SUB):
                for c in range(0, SC_BLK_COLS, _LANES):
                    accs[n & 3] = accs[n & 3] + x_vmem[pl.ds(r, 1),
                                                       pl.ds(c, _LANES)]
                    n += 1
            acc_v[...] += (accs[0] + accs[1]) + (accs[2] + accs[3])

        pltpu.emit_pipeline(
            body,
            grid=(SC_ROW_GROUPS, SC_COL_BLKS),
            in_specs=[pl.BlockSpec(
                (SUB, SC_BLK_COLS),
                lambda i, j: (TC_ROWS // SUB + i, j),
            )],
            core_axis_name=("c", "s"),
            dimension_semantics=(pltpu.PARALLEL, pltpu.ARBITRARY),
        )(x_hbm)

        pltpu.sync_copy(acc_v, p_hbm.at[wid])

    return k(output)


def _sum_body(starts_sm, x_hbm, o_ref, gout_hbm, acc_ref, bufs, sems, gsem):
    j = pl.program_id(0)

    def _issue(blk, slot):
        pltpu.make_async_copy(
            x_hbm.at[pl.ds(blk * SUM_BLK_ROWS, SUM_BLK_ROWS),
                     pl.ds(0, V_ALIGNED)],
            bufs.at[slot],
            sems.at[slot],
        ).start()

    @pl.when(j == 0)
    def _():
        acc_ref[...] = jnp.zeros((SUM_BLK_ROWS, GRAN), jnp.float32)
        for s in range(DEPTH):
            _issue(s, s)

        @pl.loop(0, B)
        def _(i):
            st = pl.multiple_of(starts_sm[i], GRAN)
            r0 = pl.multiple_of((i // SUB) * SUB, SUB)
            pltpu.make_async_copy(
                x_hbm.at[pl.ds(r0, SUB), pl.ds(st, GRAN)],
                gout_hbm.at[i], gsem,
            ).start()

    slot = lax.rem(j, DEPTH)
    pltpu.make_async_copy(
        x_hbm.at[pl.ds(0, SUM_BLK_ROWS), pl.ds(0, V_ALIGNED)],
        bufs.at[slot],
        sems.at[slot],
    ).wait()

    accs = [jnp.zeros((SUM_BLK_ROWS, GRAN), jnp.float32) for _ in range(4)]
    for i in range(N_TILES):
        accs[i & 3] = accs[i & 3] + bufs[slot, :, pl.ds(i * GRAN, GRAN)]
    acc_ref[...] += (accs[0] + accs[1]) + (accs[2] + accs[3])

    @pl.when(j + DEPTH < NSTEP)
    def _():
        _issue(j + DEPTH, slot)

    @pl.when(j == NSTEP - 1)
    def _():
        o_ref[...] = acc_ref[...]

        @pl.loop(0, B)
        def _(i):
            pltpu.make_async_copy(
                x_hbm.at[pl.ds(0, SUB), pl.ds(0, GRAN)],
                gout_hbm.at[0], gsem,
            ).wait()


def _combine_body(gp_ref, scp_ref, col0_ref, tail_ref, g_ref, t_ref, st_ref,
                  o_ref):
    t = t_ref[...]
    brow = jax.lax.broadcasted_iota(jnp.int32, (B, 1), 0)
    sub = jnp.bitwise_and(brow, SUB - 1)
    sub_iota = jax.lax.broadcasted_iota(jnp.int32, (B, SUB), 1)
    lane3 = jax.lax.broadcasted_iota(jnp.int32, (B, SUB, GRAN), 2)

    c = t - st_ref[...]
    bylane = jnp.sum(jnp.where(lane3 == c[:, :, None], g_ref[...], 0.0), axis=2)
    sel = jnp.sum(jnp.where(sub_iota == sub, bylane, 0.0), axis=1,
                  keepdims=True)

    t1 = jnp.sum(sel)
    t2 = jnp.sum(jnp.where(t != 0, sel, 0.0))
    n0 = jnp.sum(jnp.where(t == 0, 1.0, 0.0))
    c0 = jnp.sum(col0_ref[...])

    g_total = (jnp.sum(gp_ref[...]) + jnp.sum(scp_ref[...])
               + jnp.sum(tail_ref[...]))
    s32 = jnp.float32(SMOOTH)
    conf32 = jnp.float32(CONF)
    const = B * (conf32 * jnp.log(conf32) + (V - 2) * s32 * jnp.log(s32))
    o_ref[0, 0] = (const + n0 * s32 * jnp.log(s32)
                   - s32 * (g_total - c0 - t2) - conf32 * t1)


def kernel(output, target, one_hot):
    del one_hot
    starts = ((target // GRAN) * GRAN).astype(jnp.int32)

    sc_parts = _sc_sum(output)

    tail = output[:, V_ALIGNED:]
    col0 = output[:, 0:1]

    gpart, gathered = pl.pallas_call(
        _sum_body,
        grid_spec=pltpu.PrefetchScalarGridSpec(
            num_scalar_prefetch=1,
            grid=(NSTEP,),
            in_specs=[pl.BlockSpec(memory_space=pltpu.MemorySpace.HBM)],
            out_specs=[
                pl.BlockSpec((SUM_BLK_ROWS, GRAN), lambda j, s: (0, 0)),
                pl.BlockSpec(memory_space=pltpu.MemorySpace.HBM),
            ],
            scratch_shapes=[
                pltpu.VMEM((SUM_BLK_ROWS, GRAN), jnp.float32),
                pltpu.VMEM((DEPTH, SUM_BLK_ROWS, V_ALIGNED), jnp.float32),
                pltpu.SemaphoreType.DMA((DEPTH,)),
                pltpu.SemaphoreType.DMA,
            ],
        ),
        out_shape=[
            jax.ShapeDtypeStruct((SUM_BLK_ROWS, GRAN), jnp.float32),
            jax.ShapeDtypeStruct((B, SUB, GRAN), jnp.float32),
        ],
        compiler_params=pltpu.CompilerParams(dimension_semantics=("arbitrary",)),
    )(starts, output)

    out = pl.pallas_call(
        _combine_body,
        in_specs=[
            pl.BlockSpec((SUM_BLK_ROWS, GRAN), lambda: (0, 0)),
            pl.BlockSpec((_SC_WORKERS, 1, _LANES), lambda: (0, 0, 0)),
            pl.BlockSpec((B, 1), lambda: (0, 0)),
            pl.BlockSpec((B, V - V_ALIGNED), lambda: (0, 0)),
            pl.BlockSpec((B, SUB, GRAN), lambda: (0, 0, 0)),
            pl.BlockSpec((B, 1), lambda: (0, 0)),
            pl.BlockSpec((B, 1), lambda: (0, 0)),
        ],
        out_specs=pl.BlockSpec(memory_space=pltpu.SMEM),
        out_shape=jax.ShapeDtypeStruct((1, 1), jnp.float32),
    )(gpart, sc_parts, col0, tail, gathered, target.reshape(B, 1),
      starts.reshape(B, 1))
    return out[0, 0]


# speedup vs baseline: 1.1556x; 1.1556x over previous
import functools

import jax
import jax.numpy as jnp
from jax import lax
from jax.experimental import pallas as pl
from jax.experimental.pallas import tpu as pltpu
from jax.experimental.pallas import tpu_sc as plsc

LS = 0.1
V = 100000
B = 1024
CONF = 1.0 - LS
SMOOTH = LS / (V - 2)
GRAN = 128
SUB = 8
ALIGNED_LIMIT = (V // GRAN) * GRAN

NSTREAM = 4
SUM_BLK_ROWS = 8
NSTEP = B // (SUM_BLK_ROWS * NSTREAM)

_ROWS_PER_CORE = B // 2


def _sc_gather(output, starts):
    mesh = plsc.ScalarSubcoreMesh(axis_name="c", num_cores=2)

    @functools.partial(
        pl.kernel,
        mesh=mesh,
        out_type=jax.ShapeDtypeStruct((B, SUB, GRAN), jnp.float32),
        scratch_types=[
            pltpu.SMEM((_ROWS_PER_CORE,), jnp.int32),
            pltpu.SemaphoreType.DMA,
        ],
    )
    def k(out_hbm, st_hbm, g_hbm, st_sm, sem):
        cid = lax.axis_index("c")
        base = cid * _ROWS_PER_CORE
        pltpu.sync_copy(st_hbm.at[pl.ds(base, _ROWS_PER_CORE)], st_sm)

        @pl.loop(0, _ROWS_PER_CORE)
        def _(i):
            b = base + i
            r0 = pl.multiple_of((b // SUB) * SUB, SUB)
            st = pl.multiple_of(st_sm[i], GRAN)
            pltpu.async_copy(
                out_hbm.at[pl.ds(r0, SUB), pl.ds(st, GRAN)],
                g_hbm.at[b], sem,
            )

        @pl.loop(0, _ROWS_PER_CORE)
        def _(i):
            pltpu.make_async_copy(
                out_hbm.at[pl.ds(0, SUB), pl.ds(0, GRAN)],
                g_hbm.at[0], sem,
            ).wait()

    return k(output, starts)


def _sum_body(x0, x1, x2, x3, g_ref, c0_ref):
    g_ref[0, 0, 0] = (jnp.sum(x0[...]) + jnp.sum(x1[...])
                      + jnp.sum(x2[...]) + jnp.sum(x3[...]))
    c0_ref[0, 0, 0] = (jnp.sum(x0[:, 0:1]) + jnp.sum(x1[:, 0:1])
                       + jnp.sum(x2[:, 0:1]) + jnp.sum(x3[:, 0:1]))


def _combine_body(gp_ref, c0p_ref, g_ref, t_ref, st_ref, o_ref):
    t = t_ref[...]
    brow = jax.lax.broadcasted_iota(jnp.int32, (B, 1), 0)
    sub = jnp.bitwise_and(brow, SUB - 1)
    sub_iota = jax.lax.broadcasted_iota(jnp.int32, (B, SUB), 1)
    lane3 = jax.lax.broadcasted_iota(jnp.int32, (B, SUB, GRAN), 2)

    c = t - st_ref[...]
    bylane = jnp.sum(jnp.where(lane3 == c[:, :, None], g_ref[...], 0.0), axis=2)
    sel = jnp.sum(jnp.where(sub_iota == sub, bylane, 0.0), axis=1,
                  keepdims=True)

    t1 = jnp.sum(sel)
    t2 = jnp.sum(jnp.where(t != 0, sel, 0.0))
    n0 = jnp.sum(jnp.where(t == 0, 1.0, 0.0))

    def _acc(i, a):
        return a[0] + gp_ref[i, 0, 0], a[1] + c0p_ref[i, 0, 0]

    g_total, c0 = lax.fori_loop(0, NSTEP, _acc,
                                (jnp.float32(0.0), jnp.float32(0.0)))
    s32 = jnp.float32(SMOOTH)
    conf32 = jnp.float32(CONF)
    const = B * (conf32 * jnp.log(conf32) + (V - 2) * s32 * jnp.log(s32))
    o_ref[0, 0] = (const + n0 * s32 * jnp.log(s32)
                   - s32 * (g_total - c0 - t2) - conf32 * t1)


def kernel(output, target, one_hot):
    del one_hot
    starts = ((target // GRAN) * GRAN).astype(jnp.int32)

    gathered = _sc_gather(output, starts)

    gpart, c0part = pl.pallas_call(
        _sum_body,
        grid=(NSTEP,),
        in_specs=[
            pl.BlockSpec((SUM_BLK_ROWS, V),
                         functools.partial(lambda k, j: (NSTREAM * j + k, 0), k))
            for k in range(NSTREAM)
        ],
        out_specs=[
            pl.BlockSpec((1, 1, 1), lambda j: (j, 0, 0),
                         memory_space=pltpu.SMEM),
            pl.BlockSpec((1, 1, 1), lambda j: (j, 0, 0),
                         memory_space=pltpu.SMEM),
        ],
        out_shape=[
            jax.ShapeDtypeStruct((NSTEP, 1, 1), jnp.float32),
            jax.ShapeDtypeStruct((NSTEP, 1, 1), jnp.float32),
        ],
        compiler_params=pltpu.CompilerParams(dimension_semantics=("arbitrary",)),
    )(output, output, output, output)

    out = pl.pallas_call(
        _combine_body,
        in_specs=[
            pl.BlockSpec(memory_space=pltpu.SMEM),
            pl.BlockSpec(memory_space=pltpu.SMEM),
            pl.BlockSpec((B, SUB, GRAN), lambda: (0, 0, 0)),
            pl.BlockSpec((B, 1), lambda: (0, 0)),
            pl.BlockSpec((B, 1), lambda: (0, 0)),
        ],
        out_specs=pl.BlockSpec(memory_space=pltpu.SMEM),
        out_shape=jax.ShapeDtypeStruct((1, 1), jnp.float32),
    )(gpart, c0part, gathered, target.reshape(B, 1), starts.reshape(B, 1))
    return out[0, 0]
